# baseline (speedup 1.0000x reference)
import jax
import jax.numpy as jnp
from jax import lax
from jax.experimental import pallas as pl
from jax.experimental.pallas import tpu as pltpu
from jax.experimental.pallas import tpu_sc as plsc

_ROWS = 16
_COLS = 4096
_WORDS = _COLS // 4
_VECS = _WORDS // 16


def _body(words_hbm, out_hbm, w_vmem, out_vmem):
    c = lax.axis_index("c")
    s = lax.axis_index("s")
    row = c * 8 + s

    @pl.when(s < 8)
    def _():
        pltpu.sync_copy(words_hbm.at[row], w_vmem)
        iota4 = lax.iota(jnp.int32, 16) * 4
        carry = jnp.int32(0)
        for i in range(_VECS):
            w = w_vmem[pl.ds(i * 16, 16)]
            p = w * jnp.int32(0x01010101)
            t = lax.shift_right_arithmetic(p, jnp.int32(24))
            incl = plsc.cumsum(t)
            wb = (incl - t) + carry
            base = i * 64
            plsc.store_scatter(out_vmem, [iota4 + base],
                               (p & 255) + wb)
            plsc.store_scatter(out_vmem, [iota4 + (base + 1)],
                               (lax.shift_right_arithmetic(p, jnp.int32(8)) & 255) + wb)
            plsc.store_scatter(out_vmem, [iota4 + (base + 2)],
                               (lax.shift_right_arithmetic(p, jnp.int32(16)) & 255) + wb)
            plsc.store_scatter(out_vmem, [iota4 + (base + 3)],
                               t + wb)
            carry = carry + jnp.sum(t)
        pltpu.sync_copy(out_vmem, out_hbm.at[row])


_sc_cumsum = pl.kernel(
    _body,
    out_type=jax.ShapeDtypeStruct((_ROWS, _COLS), jnp.int32),
    mesh=plsc.VectorSubcoreMesh(
        core_axis_name="c", subcore_axis_name="s",
        num_cores=2, num_subcores=8,
    ),
    scratch_types=[
        pltpu.VMEM((_WORDS,), jnp.int32),
        pltpu.VMEM((_COLS,), jnp.int32),
    ],
    compiler_params=pltpu.CompilerParams(needs_layout_passes=False),
)


@jax.jit
def kernel(masks):
    u8 = masks.view(jnp.uint8)
    words = lax.bitcast_convert_type(u8.reshape(_ROWS, _WORDS, 4), jnp.int32)
    return _sc_cumsum(words)


# speedup vs baseline: 6.3188x; 6.3188x over previous
import jax
import jax.numpy as jnp
from jax import lax
from jax.experimental import pallas as pl

_ROWS = 16
_COLS = 4096
_BLK = 128
_NBLK = _COLS // _BLK


def _body(x_ref, o_ref):
    x = x_ref[...].astype(jnp.bfloat16)
    i = lax.broadcasted_iota(jnp.int32, (_BLK, _BLK), 0)
    j = lax.broadcasted_iota(jnp.int32, (_BLK, _BLK), 1)
    tri = (i <= j).astype(jnp.bfloat16)
    carry = jnp.zeros((_ROWS, 1), jnp.float32)
    for b in range(_NBLK):
        xb = lax.slice(x, (0, b * _BLK), (_ROWS, (b + 1) * _BLK))
        cb = lax.dot(xb, tri, preferred_element_type=jnp.float32)
        ob = cb + carry
        o_ref[:, b * _BLK:(b + 1) * _BLK] = ob.astype(jnp.int32)
        carry = carry + lax.slice(cb, (0, _BLK - 1), (_ROWS, _BLK))


@jax.jit
def kernel(masks):
    x8 = masks.view(jnp.int8)
    return pl.pallas_call(
        _body,
        out_shape=jax.ShapeDtypeStruct((_ROWS, _COLS), jnp.int32),
    )(x8)


# speedup vs baseline: 10.9522x; 1.7333x over previous
import jax
import jax.numpy as jnp
from jax import lax
from jax.experimental import pallas as pl
from jax.experimental.pallas import tpu as pltpu

_ROWS = 16
_COLS = 4096
_BLK = 128
_NBLK = _COLS // _BLK


def _body(x_ref, o_ref):
    x = x_ref[...].astype(jnp.bfloat16)
    i = lax.broadcasted_iota(jnp.int32, (_BLK, _BLK), 0)
    j = lax.broadcasted_iota(jnp.int32, (_BLK, _BLK), 1)
    tri = (i <= j).astype(jnp.bfloat16)
    cbs = []
    incl = []
    for b in range(_NBLK):
        xb = lax.slice(x, (0, b * _BLK), (_ROWS, (b + 1) * _BLK))
        cb = lax.dot(xb, tri, preferred_element_type=jnp.float32)
        cbs.append(cb)
        incl.append(lax.slice(cb, (0, _BLK - 1), (_ROWS, _BLK)))
    d = 1
    while d < _NBLK:
        incl = [incl[b] if b < d else incl[b] + incl[b - d]
                for b in range(_NBLK)]
        d *= 2
    for b in range(_NBLK):
        ob = cbs[b] if b == 0 else cbs[b] + incl[b - 1]
        o_ref[:, b * _BLK:(b + 1) * _BLK] = ob.astype(jnp.int32)


@jax.jit
def kernel(masks):
    x8 = masks.view(jnp.int8)
    return pl.pallas_call(
        _body,
        out_shape=jax.ShapeDtypeStruct((_ROWS, _COLS), jnp.int32),
        compiler_params=pltpu.CompilerParams(allow_input_fusion=[True]),
    )(x8)


# speedup vs baseline: 11.8811x; 1.0848x over previous
import jax
import jax.numpy as jnp
from jax import lax
from jax.experimental import pallas as pl
from jax.experimental.pallas import tpu as pltpu

_ROWS = 16
_COLS = 4096
_BLK = 128
_NBLK = _COLS // _BLK


def _body(x_ref, o_ref):
    x = x_ref[...]
    i = lax.broadcasted_iota(jnp.int32, (_BLK, _BLK), 0)
    j = lax.broadcasted_iota(jnp.int32, (_BLK, _BLK), 1)
    tri = (i <= j).astype(jnp.bfloat16)
    bpg = 8
    ngrp = _NBLK // bpg
    ones_col = jnp.ones((_BLK, 1), jnp.bfloat16)
    xcats = []
    for g in range(ngrp):
        b0 = g * bpg
        xcats.append(jnp.concatenate(
            [lax.slice(x, (0, (b0 + b) * _BLK), (_ROWS, (b0 + b + 1) * _BLK))
             for b in range(bpg)], axis=0))
    tots = [lax.dot(xc, ones_col, preferred_element_type=jnp.float32)
            for xc in xcats]
    offs = []
    group_total = None
    for g in range(ngrp):
        incl = [lax.slice(tots[g], (b * _ROWS, 0), ((b + 1) * _ROWS, 1))
                for b in range(bpg)]
        d = 1
        while d < bpg:
            incl = [incl[b] if b < d else incl[b] + incl[b - d]
                    for b in range(bpg)]
            d *= 2
        for b in range(bpg):
            off = group_total
            if b > 0:
                off = incl[b - 1] if off is None else off + incl[b - 1]
            offs.append(off)
        gt = incl[bpg - 1]
        group_total = gt if group_total is None else group_total + gt
    for g in range(ngrp):
        cg = lax.dot(xcats[g], tri, preferred_element_type=jnp.float32)
        for b in range(bpg):
            cb = lax.slice(cg, (b * _ROWS, 0), ((b + 1) * _ROWS, _BLK))
            off = offs[g * bpg + b]
            ob = cb if off is None else cb + off
            o_ref[:, (g * bpg + b) * _BLK:(g * bpg + b + 1) * _BLK] = (
                ob.astype(jnp.int32))


@jax.jit
def kernel(masks):
    x16 = masks.astype(jnp.bfloat16)
    return pl.pallas_call(
        _body,
        out_shape=jax.ShapeDtypeStruct((_ROWS, _COLS), jnp.int32),
        compiler_params=pltpu.CompilerParams(allow_input_fusion=[True]),
    )(x16)
